# Initial kernel scaffold; baseline (speedup 1.0000x reference)
import jax
import jax.numpy as jnp
from jax.experimental import pallas as pl


def kernel(atom_types, t, lattices, edge_index, edge2graph, frac_diff, node2graph, params):
    raise NotImplementedError("write your pallas kernel here")


# speedup vs baseline: 1.6993x; 1.6993x over previous
import functools

import jax
import jax.numpy as jnp
from jax import lax
from jax.experimental import pallas as pl
from jax.experimental.pallas import tpu as pltpu
from jax.experimental.pallas import tpu_sc as plsc

N = 10000
E = 160000
B = 100
H = 128
TD = 64
N_PAD = 10240
E_PAD = 163840
NBLK = 512
EBLK = 512
W = 128
SROWS = 10240
RPT = SROWS // 16

f32 = jnp.float32

@functools.cache
def _mesh():
    return plsc.VectorSubcoreMesh(core_axis_name="core", subcore_axis_name="subcore")


def _layer_norm(x, g, b):
    mu = jnp.mean(x, axis=-1, keepdims=True)
    var = jnp.mean((x - mu) ** 2, axis=-1, keepdims=True)
    return (x - mu) / jnp.sqrt(var + 1e-5) * g + b


def _dot(a, b):
    return jnp.dot(a, b, preferred_element_type=jnp.float32)


def _pre_body(at_ref, n2g_ref, t_ref, twT_ref, emb_ref, laT_ref, lbT_ref, lb_ref,
              g0_ref, b0_ref, w1aT_ref, w1bT_ref, lat_ref, w1cT_ref, eb1_ref,
              nf_ref, ln_ref, pa_ref, pb_ref):
    ioh = lax.broadcasted_iota(jnp.int32, (NBLK, 128), 1)
    oh_a = (at_ref[...] == ioh).astype(f32)
    oh_g = (n2g_ref[...] == ioh).astype(f32)
    t_emb = _dot(t_ref[...], twT_ref[...])
    embg = _dot(oh_a, emb_ref[...])
    tg = _dot(oh_g, t_emb)
    nf0 = jax.nn.silu(_dot(embg, laT_ref[...]) + _dot(tg, lbT_ref[...]) + lb_ref[...])
    ln0 = _layer_norm(nf0, g0_ref[...], b0_ref[...])
    qc = _dot(lat_ref[...], w1cT_ref[...]) + eb1_ref[...]
    qcn = _dot(oh_g, qc)
    nf_ref[...] = nf0
    ln_ref[...] = ln0
    pa_ref[...] = _dot(ln0, w1aT_ref[...]) + qcn
    pb_ref[...] = _dot(ln0, w1bT_ref[...])


def _edge_body(g0_ref, g1_ref, fd_ref, w1dT_ref, w2T_ref, eb2_ref, ef_ref):
    ef1 = jax.nn.silu(g0_ref[...] + g1_ref[...] + _dot(fd_ref[...], w1dT_ref[...]))
    ef_ref[...] = jax.nn.silu(_dot(ef1, w2T_ref[...]) + eb2_ref[...])


def _node_mid_body(nf_ref, ln_ref, a0_ref, a1_ref, c0_ref, c1_ref, n2g_ref,
                   nw1aT_ref, nw1bT_ref, nb1_ref, nw2T_ref, nb2_ref,
                   g2_ref, b2_ref, w1aT2_ref, w1bT2_ref, lat_ref, w1cT2_ref, eb12_ref,
                   nfo_ref, lno_ref, pao_ref, pbo_ref):
    agg = a0_ref[0] + a1_ref[0]
    cnt = c0_ref[0][:, 0:1] + c1_ref[0][:, 0:1]
    aggm = agg / jnp.maximum(cnt, 1.0)
    h = jax.nn.silu(_dot(ln_ref[...], nw1aT_ref[...]) + _dot(aggm, nw1bT_ref[...]) + nb1_ref[...])
    h = jax.nn.silu(_dot(h, nw2T_ref[...]) + nb2_ref[...])
    nfn = nf_ref[...] + h
    ln2 = _layer_norm(nfn, g2_ref[...], b2_ref[...])
    ioh = lax.broadcasted_iota(jnp.int32, (NBLK, 128), 1)
    oh_g = (n2g_ref[...] == ioh).astype(f32)
    qcn = _dot(oh_g, _dot(lat_ref[...], w1cT2_ref[...]) + eb12_ref[...])
    nfo_ref[...] = nfn
    lno_ref[...] = ln2
    pao_ref[...] = _dot(ln2, w1aT2_ref[...]) + qcn
    pbo_ref[...] = _dot(ln2, w1bT2_ref[...])


def _node_fin_body(nf_ref, ln_ref, a0_ref, a1_ref, c0_ref, c1_ref,
                   nw1aT_ref, nw1bT_ref, nb1_ref, nw2T_ref, nb2_ref,
                   fg_ref, fb_ref, cwT_ref, out_ref):
    agg = a0_ref[0] + a1_ref[0]
    cnt = c0_ref[0][:, 0:1] + c1_ref[0][:, 0:1]
    aggm = agg / jnp.maximum(cnt, 1.0)
    h = jax.nn.silu(_dot(ln_ref[...], nw1aT_ref[...]) + _dot(aggm, nw1bT_ref[...]) + nb1_ref[...])
    h = jax.nn.silu(_dot(h, nw2T_ref[...]) + nb2_ref[...])
    nfn = nf_ref[...] + h
    lnf = _layer_norm(nfn, fg_ref[...], fb_ref[...])
    out_ref[...] = _dot(lnf, cwT_ref[...])


def _sc_gather(pab, idx2):
    @functools.partial(
        pl.kernel,
        out_type=jax.ShapeDtypeStruct((2 * E_PAD, H), f32),
        mesh=_mesh())
    def k(tab_hbm, idx_hbm, g_hbm):
        def body(i_v, g_v):
            pltpu.sync_copy(tab_hbm.at[i_v.at[0]], g_v)

        pltpu.emit_pipeline(
            body,
            grid=(2 * E_PAD // W,),
            in_specs=[pl.BlockSpec((1, W), lambda i: (0, i))],
            out_specs=[pl.BlockSpec((W, H), lambda i: (i, 0))],
            core_axis_name=("core", "subcore"),
            dimension_semantics=(pltpu.PARALLEL,),
        )(idx_hbm, g_hbm)

    return k(pab, idx2)


def _sc_scatter_l1(ef, si3, zrows):
    NWIN = E_PAD // W
    PER = NWIN // 32
    @functools.partial(
        pl.kernel,
        out_type=(jax.ShapeDtypeStruct((2, SROWS, H), f32),
                  jax.ShapeDtypeStruct((2, SROWS, H), f32)),
        mesh=_mesh(),
        scratch_types=[pltpu.VMEM_SHARED((SROWS, H), f32),
                       pltpu.VMEM((W, H), f32),
                       pltpu.VMEM((W, H), f32),
                       pltpu.VMEM((1, W), jnp.int32),
                       pltpu.SemaphoreType.DMA,
                       pltpu.SemaphoreType.DMA])
    def k(ef_hbm, si_hbm, z_hbm, feat_hbm, cnt_hbm, acc, va, vb, ibuf, sa, sb):
        cid = lax.axis_index("core")
        sid = lax.axis_index("subcore")
        wid = cid * 16 + sid
        base = wid * PER

        pltpu.sync_copy(z_hbm.at[pl.ds(sid * RPT, RPT)],
                        acc.at[pl.ds(sid * RPT, RPT)])
        plsc.subcore_barrier()

        pltpu.async_copy(ef_hbm.at[pl.ds(base * W, W)], va, sa)

        @pl.loop(0, PER, step=2)
        def _(j):
            w = base + j
            pltpu.make_async_copy(ef_hbm.at[pl.ds(w * W, W)], va, sa).wait()
            pltpu.async_copy(ef_hbm.at[pl.ds((w + 1) * W, W)], vb, sb)
            pltpu.sync_copy(si_hbm.at[w], ibuf)
            pltpu.sync_copy(va, acc.at[ibuf.at[0]], add=True)
            pltpu.make_async_copy(ef_hbm.at[pl.ds((w + 1) * W, W)], vb, sb).wait()

            @pl.when(j + 2 < PER)
            def _():
                pltpu.async_copy(ef_hbm.at[pl.ds((w + 2) * W, W)], va, sa)

            pltpu.sync_copy(si_hbm.at[w + 1], ibuf)
            pltpu.sync_copy(vb, acc.at[ibuf.at[0]], add=True)

        plsc.subcore_barrier()
        pltpu.sync_copy(acc.at[pl.ds(sid * RPT, RPT)],
                        feat_hbm.at[cid, pl.ds(sid * RPT, RPT)])
        plsc.subcore_barrier()

        pltpu.sync_copy(z_hbm.at[pl.ds(sid * RPT, RPT)],
                        acc.at[pl.ds(sid * RPT, RPT)])

        @pl.loop(0, W)
        def _(r):
            @pl.loop(0, H, step=16)
            def _(c):
                va.at[pl.ds(r, 1), pl.ds(c, 16)][...] = jnp.ones((1, 16), f32)

        plsc.subcore_barrier()

        @pl.loop(0, PER)
        def _(j):
            pltpu.sync_copy(si_hbm.at[base + j], ibuf)
            pltpu.sync_copy(va, acc.at[ibuf.at[0]], add=True)

        plsc.subcore_barrier()
        pltpu.sync_copy(acc.at[pl.ds(sid * RPT, RPT)],
                        cnt_hbm.at[cid, pl.ds(sid * RPT, RPT)])

    return k(ef, si3, zrows)


def _sc_scatter_l2(ef, si3, zrows):
    NWIN = E_PAD // W
    PER = NWIN // 32
    @functools.partial(
        pl.kernel,
        out_type=jax.ShapeDtypeStruct((2, SROWS, H), f32),
        mesh=_mesh(),
        scratch_types=[pltpu.VMEM_SHARED((SROWS, H), f32),
                       pltpu.VMEM((W, H), f32),
                       pltpu.VMEM((1, W), jnp.int32)])
    def k(ef_hbm, si_hbm, z_hbm, out_hbm, acc, vbuf, ibuf):
        cid = lax.axis_index("core")
        sid = lax.axis_index("subcore")
        wid = cid * 16 + sid

        pltpu.sync_copy(z_hbm.at[pl.ds(sid * RPT, RPT)],
                        acc.at[pl.ds(sid * RPT, RPT)])
        plsc.subcore_barrier()

        @pl.loop(0, PER)
        def _(j):
            w = wid * PER + j
            pltpu.sync_copy(si_hbm.at[w], ibuf)
            pltpu.sync_copy(ef_hbm.at[pl.ds(w * W, W)], vbuf)
            pltpu.sync_copy(vbuf, acc.at[ibuf.at[0]], add=True)

        plsc.subcore_barrier()
        pltpu.sync_copy(acc.at[pl.ds(sid * RPT, RPT)],
                        out_hbm.at[cid, pl.ds(sid * RPT, RPT)])

    return k(ef, si3, zrows)


def _fsp(shape):
    return pl.BlockSpec(shape, lambda i: (0,) * len(shape))


def _nsp(c):
    return pl.BlockSpec((NBLK, c), lambda i: (i, 0))


def kernel(atom_types, t, lattices, edge_index, edge2graph, frac_diff, node2graph, params):
    p = params
    at2 = jnp.pad(atom_types.astype(jnp.int32), (0, N_PAD - N)).reshape(N_PAD, 1)
    n2g2 = jnp.pad(node2graph.astype(jnp.int32), (0, N_PAD - N)).reshape(N_PAD, 1)
    src = edge_index[0].astype(jnp.int32)
    dst = edge_index[1].astype(jnp.int32)
    pad_idx = jnp.full((E_PAD - E,), N, jnp.int32)
    si = jnp.concatenate([src, pad_idx]).reshape(1, E_PAD)
    di = jnp.concatenate([dst, pad_idx]).reshape(1, E_PAD)
    fd = jnp.zeros((E_PAD, 8), f32).at[:E, :3].set(frac_diff)
    t_pad = jnp.zeros((128, 1), f32).at[:B].set(t)
    lat_pad = jnp.zeros((128, 8), f32).at[:B, :6].set(lattices)
    emb_pad = jnp.zeros((128, H), f32).at[:p['emb'].shape[0]].set(p['emb'])
    twT = p['time_W'].T.reshape(1, TD)
    laT = p['latent_W'][:, :H].T
    lbT = p['latent_W'][:, H:].T
    lb = p['latent_b'].reshape(1, H)
    L = []
    for li in range(2):
        lp = p['layers'][li]
        L.append(dict(
            g=lp['ln_g'].reshape(1, H), b=lp['ln_b'].reshape(1, H),
            w1aT=lp['e_W1'][:, :H].T, w1bT=lp['e_W1'][:, H:2 * H].T,
            w1cT=jnp.zeros((8, H), f32).at[:6].set(lp['e_W1'][:, 2 * H:2 * H + 6].T),
            w1dT=jnp.zeros((8, H), f32).at[:3].set(lp['e_W1'][:, 2 * H + 6:].T),
            eb1=lp['e_b1'].reshape(1, H),
            w2T=lp['e_W2'].T, eb2=lp['e_b2'].reshape(1, H),
            nw1aT=lp['n_W1'][:, :H].T, nw1bT=lp['n_W1'][:, H:].T,
            nb1=lp['n_b1'].reshape(1, H), nw2T=lp['n_W2'].T,
            nb2=lp['n_b2'].reshape(1, H),
        ))
    fg = p['final_g'].reshape(1, H)
    fb = p['final_b'].reshape(1, H)
    cwT = jnp.zeros((H, 8), f32).at[:, :3].set(p['coord_W'].T)

    ng = N_PAD // NBLK
    eg = E_PAD // EBLK

    zrows = jnp.zeros((SROWS, H), f32)

    nf, ln, pa, pb = pl.pallas_call(
        _pre_body,
        grid=(ng,),
        in_specs=[
            pl.BlockSpec((NBLK, 1), lambda i: (i, 0)),
            pl.BlockSpec((NBLK, 1), lambda i: (i, 0)),
            _fsp((128, 1)), _fsp((1, TD)), _fsp((128, H)), _fsp((H, H)),
            _fsp((TD, H)), _fsp((1, H)),
            _fsp((1, H)), _fsp((1, H)), _fsp((H, H)), _fsp((H, H)),
            _fsp((128, 8)), _fsp((8, H)), _fsp((1, H)),
        ],
        out_specs=[_nsp(H)] * 4,
        out_shape=[jax.ShapeDtypeStruct((N_PAD, H), f32)] * 4,
    )(at2, n2g2, t_pad, twT, emb_pad, laT, lbT, lb,
      L[0]['g'], L[0]['b'], L[0]['w1aT'], L[0]['w1bT'],
      lat_pad, L[0]['w1cT'], L[0]['eb1'])

    out = None
    idx2 = jnp.concatenate([si, di + N_PAD], axis=1)
    si3 = si.reshape(E_PAD // W, 1, W)
    for li in range(2):
        gcat = _sc_gather(jnp.concatenate([pa, pb]), idx2)
        ef2 = pl.pallas_call(
            _edge_body,
            grid=(eg,),
            in_specs=[
                pl.BlockSpec((EBLK, H), lambda i: (i, 0)),
                pl.BlockSpec((EBLK, H), lambda i: (i + E_PAD // EBLK, 0)),
                pl.BlockSpec((EBLK, 8), lambda i: (i, 0)),
                _fsp((8, H)), _fsp((H, H)), _fsp((1, H)),
            ],
            out_specs=pl.BlockSpec((EBLK, H), lambda i: (i, 0)),
            out_shape=jax.ShapeDtypeStruct((E_PAD, H), f32),
        )(gcat, gcat, fd, L[li]['w1dT'], L[li]['w2T'], L[li]['eb2'])
        if li == 0:
            aggp, cnt = _sc_scatter_l1(ef2, si3, zrows)
        else:
            aggp = _sc_scatter_l2(ef2, si3, zrows)
        a0spec = pl.BlockSpec((1, NBLK, H), lambda i: (0, i, 0))
        a1spec = pl.BlockSpec((1, NBLK, H), lambda i: (1, i, 0))
        c0spec = pl.BlockSpec((1, NBLK, H), lambda i: (0, i, 0))
        c1spec = pl.BlockSpec((1, NBLK, H), lambda i: (1, i, 0))
        if li == 0:
            nf, ln, pa, pb = pl.pallas_call(
                _node_mid_body,
                grid=(ng,),
                in_specs=[
                    _nsp(H), _nsp(H), a0spec, a1spec, c0spec, c1spec,
                    pl.BlockSpec((NBLK, 1), lambda i: (i, 0)),
                    _fsp((H, H)), _fsp((H, H)), _fsp((1, H)), _fsp((H, H)), _fsp((1, H)),
                    _fsp((1, H)), _fsp((1, H)), _fsp((H, H)), _fsp((H, H)),
                    _fsp((128, 8)), _fsp((8, H)), _fsp((1, H)),
                ],
                out_specs=[_nsp(H)] * 4,
                out_shape=[jax.ShapeDtypeStruct((N_PAD, H), f32)] * 4,
            )(nf, ln, aggp, aggp, cnt, cnt, n2g2,
              L[0]['nw1aT'], L[0]['nw1bT'], L[0]['nb1'], L[0]['nw2T'], L[0]['nb2'],
              L[1]['g'], L[1]['b'], L[1]['w1aT'], L[1]['w1bT'],
              lat_pad, L[1]['w1cT'], L[1]['eb1'])
        else:
            out = pl.pallas_call(
                _node_fin_body,
                grid=(ng,),
                in_specs=[
                    _nsp(H), _nsp(H), a0spec, a1spec, c0spec, c1spec,
                    _fsp((H, H)), _fsp((H, H)), _fsp((1, H)), _fsp((H, H)), _fsp((1, H)),
                    _fsp((1, H)), _fsp((1, H)), _fsp((H, 8)),
                ],
                out_specs=_nsp(8),
                out_shape=jax.ShapeDtypeStruct((N_PAD, 8), f32),
            )(nf, ln, aggp, aggp, cnt, cnt,
              L[1]['nw1aT'], L[1]['nw1bT'], L[1]['nb1'], L[1]['nw2T'], L[1]['nb2'],
              fg, fb, cwT)

    return out[:N, :3]


# speedup vs baseline: 1.7323x; 1.0194x over previous
import functools

import jax
import jax.numpy as jnp
from jax import lax
from jax.experimental import pallas as pl
from jax.experimental.pallas import tpu as pltpu
from jax.experimental.pallas import tpu_sc as plsc

N = 10000
E = 160000
B = 100
H = 128
TD = 64
N_PAD = 10240
E_PAD = 163840
NBLK = 512
EBLK = 512
W = 128
SROWS = 10240
RPT = SROWS // 16

f32 = jnp.float32

@functools.cache
def _mesh():
    return plsc.VectorSubcoreMesh(core_axis_name="core", subcore_axis_name="subcore")


def _layer_norm(x, g, b):
    mu = jnp.mean(x, axis=-1, keepdims=True)
    var = jnp.mean((x - mu) ** 2, axis=-1, keepdims=True)
    return (x - mu) / jnp.sqrt(var + 1e-5) * g + b


def _dot(a, b):
    return jnp.dot(a, b, preferred_element_type=jnp.float32)


def _pre_body(at_ref, n2g_ref, t_ref, twT_ref, emb_ref, laT_ref, lbT_ref, lb_ref,
              g0_ref, b0_ref, w1aT_ref, w1bT_ref, lat_ref, w1cT_ref, eb1_ref,
              nf_ref, ln_ref, pa_ref, pb_ref):
    ioh = lax.broadcasted_iota(jnp.int32, (NBLK, 128), 1)
    oh_a = (at_ref[...] == ioh).astype(f32)
    oh_g = (n2g_ref[...] == ioh).astype(f32)
    t_emb = _dot(t_ref[...], twT_ref[...])
    embg = _dot(oh_a, emb_ref[...])
    tg = _dot(oh_g, t_emb)
    nf0 = jax.nn.silu(_dot(embg, laT_ref[...]) + _dot(tg, lbT_ref[...]) + lb_ref[...])
    ln0 = _layer_norm(nf0, g0_ref[...], b0_ref[...])
    qc = _dot(lat_ref[...], w1cT_ref[...]) + eb1_ref[...]
    qcn = _dot(oh_g, qc)
    nf_ref[...] = nf0
    ln_ref[...] = ln0
    pa_ref[...] = _dot(ln0, w1aT_ref[...]) + qcn
    pb_ref[...] = _dot(ln0, w1bT_ref[...])


def _edge_body(g0_ref, g1_ref, fd_ref, w1dT_ref, w2T_ref, eb2_ref, ef_ref):
    ef1 = jax.nn.silu(g0_ref[...] + g1_ref[...] + _dot(fd_ref[...], w1dT_ref[...]))
    ef_ref[...] = jax.nn.silu(_dot(ef1, w2T_ref[...]) + eb2_ref[...])


def _node_mid_body(nf_ref, ln_ref, a0_ref, a1_ref, c0_ref, c1_ref, n2g_ref,
                   nw1aT_ref, nw1bT_ref, nb1_ref, nw2T_ref, nb2_ref,
                   g2_ref, b2_ref, w1aT2_ref, w1bT2_ref, lat_ref, w1cT2_ref, eb12_ref,
                   nfo_ref, lno_ref, pao_ref, pbo_ref):
    agg = a0_ref[0] + a1_ref[0]
    cnt = c0_ref[0][:, 0:1] + c1_ref[0][:, 0:1]
    aggm = agg / jnp.maximum(cnt, 1.0)
    h = jax.nn.silu(_dot(ln_ref[...], nw1aT_ref[...]) + _dot(aggm, nw1bT_ref[...]) + nb1_ref[...])
    h = jax.nn.silu(_dot(h, nw2T_ref[...]) + nb2_ref[...])
    nfn = nf_ref[...] + h
    ln2 = _layer_norm(nfn, g2_ref[...], b2_ref[...])
    ioh = lax.broadcasted_iota(jnp.int32, (NBLK, 128), 1)
    oh_g = (n2g_ref[...] == ioh).astype(f32)
    qcn = _dot(oh_g, _dot(lat_ref[...], w1cT2_ref[...]) + eb12_ref[...])
    nfo_ref[...] = nfn
    lno_ref[...] = ln2
    pao_ref[...] = _dot(ln2, w1aT2_ref[...]) + qcn
    pbo_ref[...] = _dot(ln2, w1bT2_ref[...])


def _node_fin_body(nf_ref, ln_ref, a0_ref, a1_ref, c0_ref, c1_ref,
                   nw1aT_ref, nw1bT_ref, nb1_ref, nw2T_ref, nb2_ref,
                   fg_ref, fb_ref, cwT_ref, out_ref):
    agg = a0_ref[0] + a1_ref[0]
    cnt = c0_ref[0][:, 0:1] + c1_ref[0][:, 0:1]
    aggm = agg / jnp.maximum(cnt, 1.0)
    h = jax.nn.silu(_dot(ln_ref[...], nw1aT_ref[...]) + _dot(aggm, nw1bT_ref[...]) + nb1_ref[...])
    h = jax.nn.silu(_dot(h, nw2T_ref[...]) + nb2_ref[...])
    nfn = nf_ref[...] + h
    lnf = _layer_norm(nfn, fg_ref[...], fb_ref[...])
    out_ref[...] = _dot(lnf, cwT_ref[...])


def _sc_gather(pab, idx2):
    @functools.partial(
        pl.kernel,
        out_type=jax.ShapeDtypeStruct((2 * E_PAD, H), f32),
        mesh=_mesh())
    def k(tab_hbm, idx_hbm, g_hbm):
        def body(i_v, g_v):
            pltpu.sync_copy(tab_hbm.at[i_v.at[0]], g_v)

        pltpu.emit_pipeline(
            body,
            grid=(2 * E_PAD // W,),
            in_specs=[pl.BlockSpec((1, W), lambda i: (0, i))],
            out_specs=[pl.BlockSpec((W, H), lambda i: (i, 0))],
            core_axis_name=("core", "subcore"),
            dimension_semantics=(pltpu.PARALLEL,),
        )(idx_hbm, g_hbm)

    return k(pab, idx2)


def _sc_scatter_l1(ef, si3, zrows):
    NWIN = E_PAD // W
    PER = NWIN // 32
    @functools.partial(
        pl.kernel,
        out_type=(jax.ShapeDtypeStruct((2, SROWS, H), f32),
                  jax.ShapeDtypeStruct((2, SROWS, H), f32)),
        mesh=_mesh(),
        scratch_types=[pltpu.VMEM_SHARED((SROWS, H), f32),
                       pltpu.VMEM((W, H), f32),
                       pltpu.VMEM((W, H), f32),
                       pltpu.VMEM((1, W), jnp.int32),
                       pltpu.VMEM((1, W), jnp.int32),
                       pltpu.SemaphoreType.DMA,
                       pltpu.SemaphoreType.DMA,
                       pltpu.SemaphoreType.DMA,
                       pltpu.SemaphoreType.DMA])
    def k(ef_hbm, si_hbm, z_hbm, feat_hbm, cnt_hbm, acc,
          va, vb, i0, i1, sla, slb, ssa, ssb):
        cid = lax.axis_index("core")
        sid = lax.axis_index("subcore")
        wid = cid * 16 + sid
        base = wid * PER

        pltpu.sync_copy(z_hbm.at[pl.ds(sid * RPT, RPT)],
                        acc.at[pl.ds(sid * RPT, RPT)])
        plsc.subcore_barrier()

        pltpu.async_copy(ef_hbm.at[pl.ds(base * W, W)], va, sla)
        pltpu.sync_copy(si_hbm.at[base], i0)

        @pl.loop(0, PER, step=2)
        def _(j):
            w = base + j
            pltpu.make_async_copy(ef_hbm.at[pl.ds(w * W, W)], va, sla).wait()

            @pl.when(j > 0)
            def _():
                pltpu.make_async_copy(vb, acc.at[i1.at[0]], ssb).wait()

            pltpu.async_copy(ef_hbm.at[pl.ds((w + 1) * W, W)], vb, slb)
            sca = pltpu.async_copy(va, acc.at[i0.at[0]], ssa, add=True)
            pltpu.sync_copy(si_hbm.at[w + 1], i1)
            pltpu.make_async_copy(ef_hbm.at[pl.ds((w + 1) * W, W)], vb, slb).wait()
            sca.wait()

            @pl.when(j + 2 < PER)
            def _():
                pltpu.async_copy(ef_hbm.at[pl.ds((w + 2) * W, W)], va, sla)

            pltpu.async_copy(vb, acc.at[i1.at[0]], ssb, add=True)

            @pl.when(j + 2 < PER)
            def _():
                pltpu.sync_copy(si_hbm.at[w + 2], i0)

        pltpu.make_async_copy(vb, acc.at[i1.at[0]], ssb).wait()
        plsc.subcore_barrier()
        pltpu.sync_copy(acc.at[pl.ds(sid * RPT, RPT)],
                        feat_hbm.at[cid, pl.ds(sid * RPT, RPT)])
        plsc.subcore_barrier()

        pltpu.sync_copy(z_hbm.at[pl.ds(sid * RPT, RPT)],
                        acc.at[pl.ds(sid * RPT, RPT)])

        @pl.loop(0, W)
        def _(r):
            @pl.loop(0, H, step=16)
            def _(c):
                va.at[pl.ds(r, 1), pl.ds(c, 16)][...] = jnp.ones((1, 16), f32)

        plsc.subcore_barrier()
        pltpu.sync_copy(si_hbm.at[base], i0)

        @pl.loop(0, PER, step=2)
        def _(j):
            w = base + j
            sca = pltpu.async_copy(va, acc.at[i0.at[0]], ssa, add=True)
            pltpu.sync_copy(si_hbm.at[w + 1], i1)
            scb = pltpu.async_copy(va, acc.at[i1.at[0]], ssb, add=True)
            sca.wait()

            @pl.when(j + 2 < PER)
            def _():
                pltpu.sync_copy(si_hbm.at[w + 2], i0)

            scb.wait()

        plsc.subcore_barrier()
        pltpu.sync_copy(acc.at[pl.ds(sid * RPT, RPT)],
                        cnt_hbm.at[cid, pl.ds(sid * RPT, RPT)])

    return k(ef, si3, zrows)


def _sc_scatter_l2(ef, si3, zrows):
    NWIN = E_PAD // W
    PER = NWIN // 32
    @functools.partial(
        pl.kernel,
        out_type=jax.ShapeDtypeStruct((2, SROWS, H), f32),
        mesh=_mesh(),
        scratch_types=[pltpu.VMEM_SHARED((SROWS, H), f32),
                       pltpu.VMEM((W, H), f32),
                       pltpu.VMEM((1, W), jnp.int32),
                       pltpu.VMEM((1, W), jnp.int32),
                       pltpu.SemaphoreType.DMA,
                       pltpu.SemaphoreType.DMA])
    def k(ef_hbm, si_hbm, z_hbm, out_hbm, acc, vc, i0, i1, ssa, ssb):
        cid = lax.axis_index("core")
        sid = lax.axis_index("subcore")
        wid = cid * 16 + sid
        base = wid * PER

        pltpu.sync_copy(z_hbm.at[pl.ds(sid * RPT, RPT)],
                        acc.at[pl.ds(sid * RPT, RPT)])
        plsc.subcore_barrier()
        pltpu.sync_copy(si_hbm.at[base], i0)

        @pl.loop(0, PER, step=2)
        def _(j):
            w = base + j
            pltpu.sync_copy(ef_hbm.at[pl.ds(w * W, W)], vc)
            sca = pltpu.async_copy(vc, acc.at[i0.at[0]], ssa, add=True)
            pltpu.sync_copy(si_hbm.at[w + 1], i1)
            sca.wait()
            pltpu.sync_copy(ef_hbm.at[pl.ds((w + 1) * W, W)], vc)
            scb = pltpu.async_copy(vc, acc.at[i1.at[0]], ssb, add=True)

            @pl.when(j + 2 < PER)
            def _():
                pltpu.sync_copy(si_hbm.at[w + 2], i0)

            scb.wait()

        plsc.subcore_barrier()
        pltpu.sync_copy(acc.at[pl.ds(sid * RPT, RPT)],
                        out_hbm.at[cid, pl.ds(sid * RPT, RPT)])

    return k(ef, si3, zrows)


def _fsp(shape):
    return pl.BlockSpec(shape, lambda i: (0,) * len(shape))


def _nsp(c):
    return pl.BlockSpec((NBLK, c), lambda i: (i, 0))


def kernel(atom_types, t, lattices, edge_index, edge2graph, frac_diff, node2graph, params):
    p = params
    at2 = jnp.pad(atom_types.astype(jnp.int32), (0, N_PAD - N)).reshape(N_PAD, 1)
    n2g2 = jnp.pad(node2graph.astype(jnp.int32), (0, N_PAD - N)).reshape(N_PAD, 1)
    src = edge_index[0].astype(jnp.int32)
    dst = edge_index[1].astype(jnp.int32)
    pad_idx = jnp.full((E_PAD - E,), N, jnp.int32)
    si = jnp.concatenate([src, pad_idx]).reshape(1, E_PAD)
    di = jnp.concatenate([dst, pad_idx]).reshape(1, E_PAD)
    fd = jnp.zeros((E_PAD, 8), f32).at[:E, :3].set(frac_diff)
    t_pad = jnp.zeros((128, 1), f32).at[:B].set(t)
    lat_pad = jnp.zeros((128, 8), f32).at[:B, :6].set(lattices)
    emb_pad = jnp.zeros((128, H), f32).at[:p['emb'].shape[0]].set(p['emb'])
    twT = p['time_W'].T.reshape(1, TD)
    laT = p['latent_W'][:, :H].T
    lbT = p['latent_W'][:, H:].T
    lb = p['latent_b'].reshape(1, H)
    L = []
    for li in range(2):
        lp = p['layers'][li]
        L.append(dict(
            g=lp['ln_g'].reshape(1, H), b=lp['ln_b'].reshape(1, H),
            w1aT=lp['e_W1'][:, :H].T, w1bT=lp['e_W1'][:, H:2 * H].T,
            w1cT=jnp.zeros((8, H), f32).at[:6].set(lp['e_W1'][:, 2 * H:2 * H + 6].T),
            w1dT=jnp.zeros((8, H), f32).at[:3].set(lp['e_W1'][:, 2 * H + 6:].T),
            eb1=lp['e_b1'].reshape(1, H),
            w2T=lp['e_W2'].T, eb2=lp['e_b2'].reshape(1, H),
            nw1aT=lp['n_W1'][:, :H].T, nw1bT=lp['n_W1'][:, H:].T,
            nb1=lp['n_b1'].reshape(1, H), nw2T=lp['n_W2'].T,
            nb2=lp['n_b2'].reshape(1, H),
        ))
    fg = p['final_g'].reshape(1, H)
    fb = p['final_b'].reshape(1, H)
    cwT = jnp.zeros((H, 8), f32).at[:, :3].set(p['coord_W'].T)

    ng = N_PAD // NBLK
    eg = E_PAD // EBLK

    zrows = jnp.zeros((SROWS, H), f32)

    nf, ln, pa, pb = pl.pallas_call(
        _pre_body,
        grid=(ng,),
        in_specs=[
            pl.BlockSpec((NBLK, 1), lambda i: (i, 0)),
            pl.BlockSpec((NBLK, 1), lambda i: (i, 0)),
            _fsp((128, 1)), _fsp((1, TD)), _fsp((128, H)), _fsp((H, H)),
            _fsp((TD, H)), _fsp((1, H)),
            _fsp((1, H)), _fsp((1, H)), _fsp((H, H)), _fsp((H, H)),
            _fsp((128, 8)), _fsp((8, H)), _fsp((1, H)),
        ],
        out_specs=[_nsp(H)] * 4,
        out_shape=[jax.ShapeDtypeStruct((N_PAD, H), f32)] * 4,
    )(at2, n2g2, t_pad, twT, emb_pad, laT, lbT, lb,
      L[0]['g'], L[0]['b'], L[0]['w1aT'], L[0]['w1bT'],
      lat_pad, L[0]['w1cT'], L[0]['eb1'])

    out = None
    idx2 = jnp.concatenate([si, di + N_PAD], axis=1)
    si3 = si.reshape(E_PAD // W, 1, W)
    for li in range(2):
        gcat = _sc_gather(jnp.concatenate([pa, pb]), idx2)
        ef2 = pl.pallas_call(
            _edge_body,
            grid=(eg,),
            in_specs=[
                pl.BlockSpec((EBLK, H), lambda i: (i, 0)),
                pl.BlockSpec((EBLK, H), lambda i: (i + E_PAD // EBLK, 0)),
                pl.BlockSpec((EBLK, 8), lambda i: (i, 0)),
                _fsp((8, H)), _fsp((H, H)), _fsp((1, H)),
            ],
            out_specs=pl.BlockSpec((EBLK, H), lambda i: (i, 0)),
            out_shape=jax.ShapeDtypeStruct((E_PAD, H), f32),
        )(gcat, gcat, fd, L[li]['w1dT'], L[li]['w2T'], L[li]['eb2'])
        if li == 0:
            aggp, cnt = _sc_scatter_l1(ef2, si3, zrows)
        else:
            aggp = _sc_scatter_l2(ef2, si3, zrows)
        a0spec = pl.BlockSpec((1, NBLK, H), lambda i: (0, i, 0))
        a1spec = pl.BlockSpec((1, NBLK, H), lambda i: (1, i, 0))
        c0spec = pl.BlockSpec((1, NBLK, H), lambda i: (0, i, 0))
        c1spec = pl.BlockSpec((1, NBLK, H), lambda i: (1, i, 0))
        if li == 0:
            nf, ln, pa, pb = pl.pallas_call(
                _node_mid_body,
                grid=(ng,),
                in_specs=[
                    _nsp(H), _nsp(H), a0spec, a1spec, c0spec, c1spec,
                    pl.BlockSpec((NBLK, 1), lambda i: (i, 0)),
                    _fsp((H, H)), _fsp((H, H)), _fsp((1, H)), _fsp((H, H)), _fsp((1, H)),
                    _fsp((1, H)), _fsp((1, H)), _fsp((H, H)), _fsp((H, H)),
                    _fsp((128, 8)), _fsp((8, H)), _fsp((1, H)),
                ],
                out_specs=[_nsp(H)] * 4,
                out_shape=[jax.ShapeDtypeStruct((N_PAD, H), f32)] * 4,
            )(nf, ln, aggp, aggp, cnt, cnt, n2g2,
              L[0]['nw1aT'], L[0]['nw1bT'], L[0]['nb1'], L[0]['nw2T'], L[0]['nb2'],
              L[1]['g'], L[1]['b'], L[1]['w1aT'], L[1]['w1bT'],
              lat_pad, L[1]['w1cT'], L[1]['eb1'])
        else:
            out = pl.pallas_call(
                _node_fin_body,
                grid=(ng,),
                in_specs=[
                    _nsp(H), _nsp(H), a0spec, a1spec, c0spec, c1spec,
                    _fsp((H, H)), _fsp((H, H)), _fsp((1, H)), _fsp((H, H)), _fsp((1, H)),
                    _fsp((1, H)), _fsp((1, H)), _fsp((H, 8)),
                ],
                out_specs=_nsp(8),
                out_shape=jax.ShapeDtypeStruct((N_PAD, 8), f32),
            )(nf, ln, aggp, aggp, cnt, cnt,
              L[1]['nw1aT'], L[1]['nw1bT'], L[1]['nb1'], L[1]['nw2T'], L[1]['nb2'],
              fg, fb, cwT)

    return out[:N, :3]
